# baseline (speedup 1.0000x reference)
import functools

import jax
import jax.numpy as jnp
from jax import lax
from jax.experimental import pallas as pl
from jax.experimental.pallas import tpu as pltpu
from jax.experimental.pallas import tpu_sc as plsc

BATCH = 1024
MAXLEN = 200
EMBED = 64
LANES = 16

NUM_CORES = 2
NUM_SUBCORES = 16
NW = NUM_CORES * NUM_SUBCORES

NBG = BATCH // 128
NLQ = NW // NBG
LQ = MAXLEN // NLQ
TPAD = 129


def _body(xt_hbm, tok_hbm, pos_hbm, t5_hbm,
          idx_v, pos_v, grows0, grows1, tbuf0, tbuf1,
          g0, g1, s0, s1):
  wid = lax.axis_index("s") * NUM_CORES + lax.axis_index("c")
  bg = wid % NBG
  lq = wid // NBG
  l0 = lq * LQ

  pltpu.sync_copy(xt_hbm.at[pl.ds(l0, LQ), pl.ds(bg * 128, 128)], idx_v)
  pltpu.sync_copy(pos_hbm.at[pl.ds(l0, LQ)], pos_v)

  grows = (grows0, grows1)
  tbufs = (tbuf0, tbuf1)
  gsem = (g0, g1)
  ssem = (s0, s1)

  iota = lax.iota(jnp.int32, LANES)
  e_idx = [iota + LANES * j for j in range(EMBED // LANES)]

  def start_gather(i, p):
    pltpu.async_copy(tok_hbm.at[idx_v.at[i]], grows[p], gsem[p])

  def wait_gather(p):
    pltpu.make_async_copy(tok_hbm.at[pl.ds(0, 128)], grows[p], gsem[p]).wait()

  def drain_stores(p):
    pltpu.make_async_copy(tok_hbm.at[pl.ds(0, 64), pl.ds(0, 128)],
                          tbufs[p].at[pl.ds(0, EMBED), pl.ds(0, 128)],
                          ssem[p]).wait()

  start_gather(0, 0)
  start_gather(1, 1)

  def step(i, p):
    wait_gather(p)

    @pl.when(i + 2 < LQ)
    def _():
      start_gather(i + 2, p)

    @pl.when(i >= 2)
    def _():
      drain_stores(p)

    pvec = [pos_v[i, pl.ds(LANES * j, LANES)] for j in range(EMBED // LANES)]

    @plsc.parallel_loop(0, 128, 1, unroll=2)
    def _(b):
      bvec = jnp.full((LANES,), b, jnp.int32)
      for j in range(EMBED // LANES):
        val = grows[p][b, pl.ds(LANES * j, LANES)] + pvec[j]
        plsc.store_scatter(tbufs[p], [e_idx[j], bvec], val)

    for eh in range(EMBED // 8):
      pltpu.async_copy(
          tbufs[p].at[pl.ds(8 * eh, 8), pl.ds(0, 128)],
          t5_hbm.at[l0 + i, eh, bg], ssem[p])

  def pair(i0, _):
    step(i0, 0)
    step(i0 + 1, 1)
    return 0

  lax.fori_loop(0, LQ // 2, lambda k, s: pair(k * 2, s), 0, unroll=False)
  drain_stores(0)
  drain_stores(1)


@jax.jit
def _tok_pos_embed(xt, token_table, pos_table):
  mesh = plsc.VectorSubcoreMesh(core_axis_name="c", subcore_axis_name="s")
  kern = functools.partial(
      pl.kernel,
      out_type=jax.ShapeDtypeStruct((MAXLEN, 8, NBG, 8, 128), jnp.float32),
      mesh=mesh,
      scratch_types=[
          pltpu.VMEM((LQ, 128), jnp.int32),
          pltpu.VMEM((LQ, EMBED), jnp.float32),
          pltpu.VMEM((128, 2 * EMBED), jnp.float32),
          pltpu.VMEM((128, 2 * EMBED), jnp.float32),
          pltpu.VMEM((EMBED, TPAD), jnp.float32),
          pltpu.VMEM((EMBED, TPAD), jnp.float32),
          pltpu.SemaphoreType.DMA,
          pltpu.SemaphoreType.DMA,
          pltpu.SemaphoreType.DMA,
          pltpu.SemaphoreType.DMA,
      ],
      compiler_params=pltpu.CompilerParams(
          use_tc_tiling_on_sc=False, needs_layout_passes=False),
  )(_body)
  return kern(xt, token_table, pos_table)


def kernel(x, token_table, pos_table):
  tok_pad = jnp.pad(token_table, ((0, 0), (0, EMBED)))
  t5 = _tok_pos_embed(x.T.astype(jnp.int32), tok_pad, pos_table)
  return t5.transpose(2, 4, 0, 1, 3).reshape(BATCH, MAXLEN, EMBED)


# speedup vs baseline: 1.0839x; 1.0839x over previous
import functools

import jax
import jax.numpy as jnp
from jax import lax
from jax.experimental import pallas as pl
from jax.experimental.pallas import tpu as pltpu
from jax.experimental.pallas import tpu_sc as plsc

BATCH = 1024
MAXLEN = 200
EMBED = 64
LANES = 16

NUM_CORES = 2
NUM_SUBCORES = 16
NW = NUM_CORES * NUM_SUBCORES

NBG = BATCH // 128
NLQ = NW // NBG
LQ = MAXLEN // NLQ
TPAD = 129


def _body(xt_hbm, tok_hbm, pos_hbm, t5_hbm,
          idx_v, pos_v, grows0, grows1, tbuf0, tbuf1,
          g0, g1, s0, s1):
  wid = lax.axis_index("s") * NUM_CORES + lax.axis_index("c")
  bg = wid % NBG
  lq = wid // NBG
  l0 = lq * LQ

  pltpu.sync_copy(xt_hbm.at[pl.ds(l0, LQ), pl.ds(bg * 128, 128)], idx_v)
  pltpu.sync_copy(pos_hbm.at[pl.ds(l0, LQ)], pos_v)

  grows = (grows0, grows1)
  tbufs = (tbuf0, tbuf1)
  gsem = (g0, g1)
  ssem = (s0, s1)

  iota = lax.iota(jnp.int32, LANES)
  e_idx = [iota + LANES * j for j in range(EMBED // LANES)]

  def start_gather(i, p):
    pltpu.async_copy(tok_hbm.at[idx_v.at[i]], grows[p], gsem[p])

  def wait_gather(p):
    pltpu.make_async_copy(tok_hbm.at[pl.ds(0, 128)], grows[p], gsem[p]).wait()

  def drain_stores(p):
    pltpu.make_async_copy(tok_hbm.at[pl.ds(0, 128)], grows[p], ssem[p]).wait()

  start_gather(0, 0)
  start_gather(1, 1)

  def step(i, p):
    wait_gather(p)

    @pl.when(i + 2 < LQ)
    def _():
      start_gather(i + 2, p)

    @pl.when(i >= 2)
    def _():
      drain_stores(p)

    pvec = [pos_v[i, pl.ds(LANES * j, LANES)] for j in range(EMBED // LANES)]

    @plsc.parallel_loop(0, 128, 1, unroll=4)
    def _(b):
      bvec = jnp.full((LANES,), b, jnp.int32)
      for j in range(EMBED // LANES):
        val = grows[p][b, pl.ds(LANES * j, LANES)] + pvec[j]
        plsc.store_scatter(tbufs[p], [e_idx[j], bvec], val)

    for eh in range(EMBED // 8):
      pltpu.async_copy(
          tbufs[p].at[pl.ds(8 * eh, 8), pl.ds(0, 128)],
          t5_hbm.at[l0 + i, eh, bg], ssem[p])

  def pair(i0, _):
    step(i0, 0)
    step(i0 + 1, 1)
    return 0

  lax.fori_loop(0, LQ // 2, lambda k, s: pair(k * 2, s), 0, unroll=False)
  drain_stores(0)
  drain_stores(1)


@jax.jit
def _tok_pos_embed(xt, token_table, pos_table):
  mesh = plsc.VectorSubcoreMesh(core_axis_name="c", subcore_axis_name="s")
  kern = functools.partial(
      pl.kernel,
      out_type=jax.ShapeDtypeStruct((MAXLEN, 8, NBG, 8, 128), jnp.float32),
      mesh=mesh,
      scratch_types=[
          pltpu.VMEM((LQ, 128), jnp.int32),
          pltpu.VMEM((LQ, EMBED), jnp.float32),
          pltpu.VMEM((128, EMBED), jnp.float32),
          pltpu.VMEM((128, EMBED), jnp.float32),
          pltpu.VMEM((EMBED, TPAD), jnp.float32),
          pltpu.VMEM((EMBED, TPAD), jnp.float32),
          pltpu.SemaphoreType.DMA,
          pltpu.SemaphoreType.DMA,
          pltpu.SemaphoreType.DMA,
          pltpu.SemaphoreType.DMA,
      ],
      compiler_params=pltpu.CompilerParams(
          use_tc_tiling_on_sc=False, needs_layout_passes=False),
  )(_body)
  return kern(xt, token_table, pos_table)


def kernel(x, token_table, pos_table):
  t5 = _tok_pos_embed(x.T.astype(jnp.int32), token_table, pos_table)
  return t5.transpose(2, 4, 0, 1, 3).reshape(BATCH, MAXLEN, EMBED)


# speedup vs baseline: 1.0932x; 1.0086x over previous
import functools

import jax
import jax.numpy as jnp
from jax import lax
from jax.experimental import pallas as pl
from jax.experimental.pallas import tpu as pltpu
from jax.experimental.pallas import tpu_sc as plsc

BATCH = 1024
MAXLEN = 200
EMBED = 64
LANES = 16

NUM_CORES = 2
NUM_SUBCORES = 16
NW = NUM_CORES * NUM_SUBCORES

NBG = BATCH // 128
NLQ = NW // NBG
LQ = MAXLEN // NLQ
LPG = 2
NG = LQ // LPG
TPAD = 129


def _body(xt_hbm, tok_hbm, pos_hbm, t5_hbm,
          idx_v, pos_v, grows0, grows1, tbuf0, tbuf1,
          g0, g1, s0, s1):
  wid = lax.axis_index("s") * NUM_CORES + lax.axis_index("c")
  bg = wid % NBG
  lq = wid // NBG
  l0 = lq * LQ

  pltpu.sync_copy(xt_hbm.at[pl.ds(l0, LQ), pl.ds(bg * 128, 128)], idx_v)
  pltpu.sync_copy(pos_hbm.at[pl.ds(l0, LQ)], pos_v)

  grows = (grows0, grows1)
  tbufs = (tbuf0, tbuf1)
  gsem = (g0, g1)
  ssem = (s0, s1)

  iota = lax.iota(jnp.int32, LANES)
  e_idx = [iota + LANES * j for j in range(EMBED // LANES)]

  def start_gather(g, p):
    for s in range(LPG):
      pltpu.async_copy(
          tok_hbm.at[idx_v.at[g * LPG + s]],
          grows[p].at[pl.ds(128 * s, 128)], gsem[p])

  def wait_gather(p):
    pltpu.make_async_copy(
        tok_hbm.at[pl.ds(0, LPG * 128)], grows[p], gsem[p]).wait()

  def drain_stores(p):
    pltpu.make_async_copy(
        tok_hbm.at[pl.ds(0, LPG * 128)], grows[p], ssem[p]).wait()

  start_gather(0, 0)
  start_gather(1, 1)

  def step(g, p):
    wait_gather(p)

    @pl.when(g + 2 < NG)
    def _():
      start_gather(g + 2, p)

    @pl.when(g >= 2)
    def _():
      drain_stores(p)

    for s in range(LPG):
      i = g * LPG + s
      pvec = [pos_v[i, pl.ds(LANES * j, LANES)]
              for j in range(EMBED // LANES)]

      @plsc.parallel_loop(0, 128, 1, unroll=4)
      def _(b):
        bvec = jnp.full((LANES,), b, jnp.int32)
        for j in range(EMBED // LANES):
          val = grows[p][128 * s + b, pl.ds(LANES * j, LANES)] + pvec[j]
          plsc.store_scatter(tbufs[p], [e_idx[j], s * 136 + bvec], val)

      for eh in range(EMBED // 8):
        pltpu.async_copy(
            tbufs[p].at[pl.ds(8 * eh, 8), pl.ds(136 * s, 128)],
            t5_hbm.at[l0 + i, eh, bg], ssem[p])

  def pair(g0_, _):
    step(g0_, 0)
    step(g0_ + 1, 1)
    return 0

  lax.fori_loop(0, NG // 2, lambda k, s: pair(k * 2, s), 0, unroll=False)
  step(NG - 1, 0)
  drain_stores(1)
  drain_stores(0)


@jax.jit
def _tok_pos_embed(xt, token_table, pos_table):
  mesh = plsc.VectorSubcoreMesh(core_axis_name="c", subcore_axis_name="s")
  kern = functools.partial(
      pl.kernel,
      out_type=jax.ShapeDtypeStruct((MAXLEN, 8, NBG, 8, 128), jnp.float32),
      mesh=mesh,
      scratch_types=[
          pltpu.VMEM((LQ, 128), jnp.int32),
          pltpu.VMEM((LQ, EMBED), jnp.float32),
          pltpu.VMEM((LPG * 128, EMBED), jnp.float32),
          pltpu.VMEM((LPG * 128, EMBED), jnp.float32),
          pltpu.VMEM((EMBED, 273), jnp.float32),
          pltpu.VMEM((EMBED, 273), jnp.float32),
          pltpu.SemaphoreType.DMA,
          pltpu.SemaphoreType.DMA,
          pltpu.SemaphoreType.DMA,
          pltpu.SemaphoreType.DMA,
      ],
      compiler_params=pltpu.CompilerParams(
          use_tc_tiling_on_sc=False, needs_layout_passes=False),
  )(_body)
  return kern(xt, token_table, pos_table)


def kernel(x, token_table, pos_table):
  t5 = _tok_pos_embed(x.T.astype(jnp.int32), token_table, pos_table)
  return t5.transpose(2, 4, 0, 1, 3).reshape(BATCH, MAXLEN, EMBED)
